# Initial kernel scaffold; baseline (speedup 1.0000x reference)
import jax
import jax.numpy as jnp
from jax.experimental import pallas as pl


def kernel(x, conv_Wr, conv_Wn, conv_b, bn_gamma, bn_beta, gs_W1, gs_b1, gs_W2, gs_b2, gh_W1, gh_b1, gh_W2, gh_b2, gh_W3, gh_b3, nh_W1, nh_b1, nh_W2, nh_b2, nh_W3, nh_b3, edge_index, batch):
    raise NotImplementedError("write your pallas kernel here")


# speedup vs baseline: 7.1967x; 7.1967x over previous
import functools

import jax
import jax.numpy as jnp
from jax import lax
from jax.experimental import pallas as pl
from jax.experimental.pallas import tpu as pltpu
from jax.experimental.pallas import tpu_sc as plsc

N = 10000
E = 320000
D = 128
B = 100
NN = 100
DS = 64
DH = 64

NC, NS = 2, 16
NW = NC * NS
EW = E // NW

NPAD = 10240
RPS = NPAD // NS

_f32 = jnp.float32


def _make_sc_agg(with_deg, ch):
    mesh = plsc.VectorSubcoreMesh(core_axis_name="c", subcore_axis_name="s")
    nfull = EW // ch
    tail = EW - nfull * ch

    out_type = [jax.ShapeDtypeStruct((NC * NPAD, D), _f32)]
    scratch = [
        pltpu.VMEM((ch,), jnp.int32),
        pltpu.VMEM((ch,), jnp.int32),
        pltpu.VMEM((ch,), jnp.int32),
        pltpu.VMEM((ch,), jnp.int32),
        pltpu.VMEM((ch, D), _f32),
        pltpu.VMEM((ch, D), _f32),
        pltpu.VMEM((tail,), jnp.int32),
        pltpu.VMEM((tail,), jnp.int32),
        pltpu.VMEM((tail, D), _f32),
        pltpu.VMEM_SHARED((NPAD, D), _f32),
        pltpu.SemaphoreType.DMA,
        pltpu.SemaphoreType.DMA,
    ]
    if with_deg:
        out_type.append(jax.ShapeDtypeStruct((NC, NPAD), _f32))
        scratch += [
            pltpu.VMEM((NPAD,), _f32),
            pltpu.VMEM((RPS,), _f32),
            pltpu.VMEM((RPS,), _f32),
            pltpu.VMEM_SHARED((NS, NPAD), _f32),
        ]

    @functools.partial(
        pl.kernel, out_type=tuple(out_type), mesh=mesh,
        scratch_types=scratch,
        compiler_params=pltpu.CompilerParams(needs_layout_passes=False))
    def agg(h_hbm, src_hbm, dst_hbm, zeros_hbm, *rest):
        if with_deg:
            (out_hbm, deg_hbm, sidx0, sidx1, didx0, didx1, rows0, rows1,
             tsidx, tdidx, trows, acc, sem0, sem1,
             hist, drow, degv, dstage) = rest
        else:
            (out_hbm, sidx0, sidx1, didx0, didx1, rows0, rows1,
             tsidx, tdidx, trows, acc, sem0, sem1) = rest

        c = lax.axis_index("c")
        s = lax.axis_index("s")
        wid = s * NC + c
        base = wid * EW

        pltpu.sync_copy(zeros_hbm.at[pl.ds(s * RPS, RPS)],
                        acc.at[pl.ds(s * RPS, RPS)])

        if with_deg:
            def zbody(j, carry):
                hist[pl.ds(j * 16, 16)] = jnp.zeros((16,), _f32)
                return carry
            lax.fori_loop(0, NPAD // 16, zbody, 0)
            ones = jnp.ones((16,), _f32)

        plsc.subcore_barrier()

        def count(idx_buf):
            if with_deg:
                for k in range(ch // 16):
                    dv = idx_buf[pl.ds(k * 16, 16)]
                    plsc.addupdate_scatter(hist, [dv], ones)

        pltpu.sync_copy(src_hbm.at[pl.ds(base, ch)], sidx0)
        pltpu.async_copy(h_hbm.at[sidx0], rows0, sem0)

        def body(i, carry):
            off0 = base + 2 * i * ch
            off1 = off0 + ch
            pltpu.sync_copy(src_hbm.at[pl.ds(off1, ch)], sidx1)
            pltpu.async_copy(h_hbm.at[sidx1], rows1, sem1)
            pltpu.sync_copy(dst_hbm.at[pl.ds(off0, ch)], didx0)
            count(didx0)
            pltpu.make_async_copy(h_hbm.at[sidx0], rows0, sem0).wait()
            pltpu.sync_copy(rows0, acc.at[didx0], add=True)
            @pl.when(2 * i + 2 < nfull)
            def _():
                pltpu.sync_copy(src_hbm.at[pl.ds(off1 + ch, ch)], sidx0)
                pltpu.async_copy(h_hbm.at[sidx0], rows0, sem0)
            pltpu.sync_copy(dst_hbm.at[pl.ds(off1, ch)], didx1)
            count(didx1)
            pltpu.make_async_copy(h_hbm.at[sidx1], rows1, sem1).wait()
            pltpu.sync_copy(rows1, acc.at[didx1], add=True)
            return carry

        lax.fori_loop(0, nfull // 2, body, 0)

        if tail:
            toff = base + nfull * ch
            pltpu.sync_copy(src_hbm.at[pl.ds(toff, tail)], tsidx)
            pltpu.sync_copy(dst_hbm.at[pl.ds(toff, tail)], tdidx)
            if with_deg:
                for k in range(tail // 16):
                    dv = tdidx[pl.ds(k * 16, 16)]
                    plsc.addupdate_scatter(hist, [dv], ones)
            pltpu.async_copy(h_hbm.at[tsidx], trows, sem0).wait()
            pltpu.sync_copy(trows, acc.at[tdidx], add=True)

        if with_deg:
            pltpu.sync_copy(hist, dstage.at[s])
            plsc.subcore_barrier()

            def zdeg(j, carry):
                degv[pl.ds(j * 16, 16)] = jnp.zeros((16,), _f32)
                return carry
            lax.fori_loop(0, RPS // 16, zdeg, 0)
            for r in range(NS):
                pltpu.sync_copy(dstage.at[r, pl.ds(s * RPS, RPS)], drow)

                def dbody(j, carry):
                    col = j * 16
                    degv[pl.ds(col, 16)] += drow[pl.ds(col, 16)]
                    return carry
                lax.fori_loop(0, RPS // 16, dbody, 0)
            pltpu.sync_copy(degv, deg_hbm.at[c, pl.ds(s * RPS, RPS)])

        plsc.subcore_barrier()
        pltpu.sync_copy(
            acc.at[pl.ds(s * RPS, RPS)],
            out_hbm.at[pl.ds(c * NPAD + s * RPS, RPS)])

    return agg


_sc_agg_deg = _make_sc_agg(True, 64)
_sc_agg = _make_sc_agg(False, 128)

BS = 400
NBLK = N // BS


def _convA_body(h_ref, part_ref, deg_ref, wr_ref, wn_ref, b_ref,
                hpre_ref, sum_ref, ssq_ref):
    i = pl.program_id(0)
    agg = part_ref[0] + part_ref[1]
    mean = agg / jnp.maximum(deg_ref[...], 1.0)
    hp = (jnp.dot(h_ref[...], wr_ref[...], preferred_element_type=_f32)
          + jnp.dot(mean, wn_ref[...], preferred_element_type=_f32)
          + b_ref[...])
    hpre_ref[...] = hp

    @pl.when(i == 0)
    def _():
        sum_ref[...] = jnp.zeros_like(sum_ref)
        ssq_ref[...] = jnp.zeros_like(ssq_ref)

    sum_ref[...] += jnp.sum(hp, axis=0, keepdims=True)
    ssq_ref[...] += jnp.sum(hp * hp, axis=0, keepdims=True)


def _convA(h, part, deg, wr, wn, b):
    return pl.pallas_call(
        _convA_body,
        grid=(NBLK,),
        in_specs=[
            pl.BlockSpec((BS, D), lambda i: (i, 0)),
            pl.BlockSpec((NC, BS, D), lambda i: (0, i, 0)),
            pl.BlockSpec((BS, 1), lambda i: (i, 0)),
            pl.BlockSpec((D, D), lambda i: (0, 0)),
            pl.BlockSpec((D, D), lambda i: (0, 0)),
            pl.BlockSpec((1, D), lambda i: (0, 0)),
        ],
        out_specs=[
            pl.BlockSpec((BS, D), lambda i: (i, 0)),
            pl.BlockSpec((1, D), lambda i: (0, 0)),
            pl.BlockSpec((1, D), lambda i: (0, 0)),
        ],
        out_shape=[
            jax.ShapeDtypeStruct((N, D), _f32),
            jax.ShapeDtypeStruct((1, D), _f32),
            jax.ShapeDtypeStruct((1, D), _f32),
        ],
    )(h, part, deg, wr, wn, b)


def _convB_body(hpre_ref, sum_ref, ssq_ref, gamma_ref, beta_ref, out_ref):
    mu = sum_ref[...] / N
    var = ssq_ref[...] / N - mu * mu
    rstd = lax.rsqrt(var + 1e-5)
    hb = (hpre_ref[...] - mu) * (rstd * gamma_ref[...]) + beta_ref[...]
    out_ref[...] = jnp.maximum(hb, 0.0)


def _convB(hpre, sm, sq, gamma, beta):
    return pl.pallas_call(
        _convB_body,
        grid=(NBLK,),
        in_specs=[
            pl.BlockSpec((BS, D), lambda i: (i, 0)),
            pl.BlockSpec((1, D), lambda i: (0, 0)),
            pl.BlockSpec((1, D), lambda i: (0, 0)),
            pl.BlockSpec((1, D), lambda i: (0, 0)),
            pl.BlockSpec((1, D), lambda i: (0, 0)),
        ],
        out_specs=pl.BlockSpec((BS, D), lambda i: (i, 0)),
        out_shape=jax.ShapeDtypeStruct((N, D), _f32),
    )(hpre, sm, sq, gamma, beta)


def _head_body(xn_ref, w1_ref, b1_ref, w2_ref, b2_ref, w3_ref, b3_ref,
               gsw1_ref, gsb1_ref, gsw2_ref, gsb2_ref,
               ghw1_ref, ghb1_ref, ghw2_ref, ghb2_ref, ghw3_ref, ghb3_ref,
               gout_ref, nout_ref):
    xn = xn_ref[...]
    h1 = lax.dot_general(xn, w1_ref[...], (((2,), (1,)), ((0,), (0,))),
                         preferred_element_type=_f32)
    h1 = jnp.maximum(h1 + b1_ref[...][:, None, :], 0.0)
    h2 = lax.dot_general(h1, w2_ref[...], (((2,), (1,)), ((0,), (0,))),
                         preferred_element_type=_f32)
    h2 = jnp.maximum(h2 + b2_ref[...][:, None, :], 0.0)
    w3 = w3_ref[...][:, :, 0]
    nout_ref[...] = jnp.sum(h2 * w3[:, None, :], axis=2) + b3_ref[...]

    g = jnp.maximum(jnp.sum(xn, axis=0) / NN, 0.0)
    g = jnp.dot(g, gsw1_ref[...], preferred_element_type=_f32) + gsb1_ref[...]
    g = jnp.dot(g, gsw2_ref[...], preferred_element_type=_f32) + gsb2_ref[...]
    g = jnp.maximum(g, 0.0)
    g = jnp.maximum(
        jnp.dot(g, ghw1_ref[...], preferred_element_type=_f32) + ghb1_ref[...], 0.0)
    g = jnp.maximum(
        jnp.dot(g, ghw2_ref[...], preferred_element_type=_f32) + ghb2_ref[...], 0.0)
    gout_ref[...] = (jnp.dot(g, ghw3_ref[...], preferred_element_type=_f32)
                     + ghb3_ref[...])


def _head(xnT, nh_W1, nh_b1, nh_W2, nh_b2, nh_W3, nh_b3,
          gs_W1, gs_b1, gs_W2, gs_b2,
          gh_W1, gh_b1, gh_W2, gh_b2, gh_W3, gh_b3):
    return pl.pallas_call(
        _head_body,
        out_shape=[
            jax.ShapeDtypeStruct((B, 1), _f32),
            jax.ShapeDtypeStruct((NN, B), _f32),
        ],
    )(xnT, nh_W1, nh_b1, nh_W2, nh_b2, nh_W3, nh_b3,
      gs_W1, gs_b1, gs_W2, gs_b2, gh_W1, gh_b1, gh_W2, gh_b2, gh_W3, gh_b3)


def kernel(x, conv_Wr, conv_Wn, conv_b, bn_gamma, bn_beta,
           gs_W1, gs_b1, gs_W2, gs_b2,
           gh_W1, gh_b1, gh_W2, gh_b2, gh_W3, gh_b3,
           nh_W1, nh_b1, nh_W2, nh_b2, nh_W3, nh_b3,
           edge_index, batch):
    src = edge_index[0]
    dst = edge_index[1]
    zeros = jnp.zeros((NPAD, D), _f32)

    part0_flat, degp = _sc_agg_deg(x, src, dst, zeros)
    part0 = part0_flat.reshape(NC, NPAD, D)
    deg = (degp[0] + degp[1])[:N].reshape(N, 1)
    hpre0, sm0, sq0 = _convA(x, part0, deg, conv_Wr[0], conv_Wn[0],
                             conv_b[0].reshape(1, D))
    h1 = _convB(hpre0, sm0, sq0, bn_gamma[0:1], bn_beta[0:1])

    part1 = _sc_agg(h1, src, dst, zeros)[0].reshape(NC, NPAD, D)
    hpre1, sm1, sq1 = _convA(h1, part1, deg, conv_Wr[1], conv_Wn[1],
                             conv_b[1].reshape(1, D))
    h2 = _convB(hpre1, sm1, sq1, bn_gamma[1:2], bn_beta[1:2])

    xnT = h2.reshape(B, NN, D).transpose(1, 0, 2)
    g_out, n_outT = _head(
        xnT, nh_W1, nh_b1, nh_W2, nh_b2, nh_W3, nh_b3,
        gs_W1, gs_b1.reshape(1, DS), gs_W2, gs_b2.reshape(1, DS),
        gh_W1, gh_b1.reshape(1, DH), gh_W2, gh_b2.reshape(1, DH),
        gh_W3, gh_b3.reshape(1, 1))
    return jnp.concatenate([g_out, n_outT.T], axis=1)


# speedup vs baseline: 8.7593x; 1.2171x over previous
import functools

import jax
import jax.numpy as jnp
from jax import lax
from jax.experimental import pallas as pl
from jax.experimental.pallas import tpu as pltpu
from jax.experimental.pallas import tpu_sc as plsc

N = 10000
E = 320000
D = 128
B = 100
NN = 100
DS = 64
DH = 64

NC, NS = 2, 16
NW = NC * NS
EW = E // NW
CH = 128
NFULL = EW // CH
TAIL = EW - NFULL * CH

NPAD = 10240
RPS = NPAD // NS

_f32 = jnp.float32


def _make_sc_agg(with_deg):
    mesh = plsc.VectorSubcoreMesh(core_axis_name="c", subcore_axis_name="s")

    out_type = [jax.ShapeDtypeStruct((NC * NPAD, D), _f32)]
    scratch = [
        pltpu.VMEM((CH,), jnp.int32),
        pltpu.VMEM((CH,), jnp.int32),
        pltpu.VMEM((CH,), jnp.int32),
        pltpu.VMEM((CH,), jnp.int32),
        pltpu.VMEM((CH, D), _f32),
        pltpu.VMEM((CH, D), _f32),
        pltpu.VMEM((TAIL,), jnp.int32),
        pltpu.VMEM((TAIL,), jnp.int32),
        pltpu.VMEM((TAIL, D), _f32),
        pltpu.VMEM_SHARED((NPAD, D), _f32),
        pltpu.SemaphoreType.DMA,
        pltpu.SemaphoreType.DMA,
        pltpu.SemaphoreType.DMA,
        pltpu.SemaphoreType.DMA,
    ]
    if with_deg:
        out_type.append(jax.ShapeDtypeStruct((NC, NPAD), _f32))
        out_type.append(jax.ShapeDtypeStruct((NW, NPAD), _f32))
        scratch += [
            pltpu.VMEM((NPAD,), _f32),
            pltpu.VMEM((RPS,), _f32),
            pltpu.VMEM((RPS,), _f32),
        ]

    @functools.partial(
        pl.kernel, out_type=tuple(out_type), mesh=mesh,
        scratch_types=scratch,
        compiler_params=pltpu.CompilerParams(needs_layout_passes=False))
    def agg(h_hbm, src_hbm, dst_hbm, *rest):
        if with_deg:
            (out_hbm, deg_hbm, stage_hbm,
             sidx0, sidx1, didx0, didx1, rows0, rows1,
             tsidx, tdidx, trows, acc, gsem0, gsem1, ssem0, ssem1,
             hist, drow, degv) = rest
        else:
            (out_hbm, sidx0, sidx1, didx0, didx1, rows0, rows1,
             tsidx, tdidx, trows, acc, gsem0, gsem1, ssem0, ssem1) = rest

        c = lax.axis_index("c")
        s = lax.axis_index("s")
        wid = s * NC + c
        base = wid * EW

        def zrow(j, carry):
            idx = j * 16
            row = idx // D
            colg = idx % D
            rows0[row, pl.ds(colg, 16)] = jnp.zeros((16,), _f32)
            return carry
        lax.fori_loop(0, CH * D // 16, zrow, 0)
        for k in range(RPS // CH):
            pltpu.sync_copy(rows0, acc.at[pl.ds(s * RPS + k * CH, CH)])

        if with_deg:
            def zhist(j, carry):
                hist[pl.ds(j * 16, 16)] = jnp.zeros((16,), _f32)
                return carry
            lax.fori_loop(0, NPAD // 16, zhist, 0)
            ones = jnp.ones((16,), _f32)

        plsc.subcore_barrier()

        def count(idx_buf):
            if with_deg:
                for k in range(CH // 16):
                    dv = idx_buf[pl.ds(k * 16, 16)]
                    plsc.addupdate_scatter(hist, [dv], ones)

        pltpu.sync_copy(src_hbm.at[pl.ds(base, CH)], sidx0)
        pltpu.async_copy(h_hbm.at[sidx0], rows0, gsem0)
        pltpu.sync_copy(src_hbm.at[pl.ds(base + CH, CH)], sidx1)
        pltpu.async_copy(h_hbm.at[sidx1], rows1, gsem1)

        def body(i, carry):
            j0 = base + 2 * i * CH
            j1 = j0 + CH
            pltpu.sync_copy(dst_hbm.at[pl.ds(j0, CH)], didx0)
            count(didx0)
            pltpu.make_async_copy(h_hbm.at[sidx0], rows0, gsem0).wait()
            pltpu.async_copy(rows0, acc.at[didx0], ssem0, add=True)
            pltpu.sync_copy(dst_hbm.at[pl.ds(j1, CH)], didx1)
            count(didx1)
            pltpu.make_async_copy(h_hbm.at[sidx1], rows1, gsem1).wait()
            pltpu.async_copy(rows1, acc.at[didx1], ssem1, add=True)
            @pl.when(2 * i + 2 < NFULL)
            def _():
                pltpu.make_async_copy(rows0, acc.at[didx0], ssem0).wait()
                pltpu.sync_copy(src_hbm.at[pl.ds(j1 + CH, CH)], sidx0)
                pltpu.async_copy(h_hbm.at[sidx0], rows0, gsem0)

            @pl.when(2 * i + 3 < NFULL)
            def _():
                pltpu.make_async_copy(rows1, acc.at[didx1], ssem1).wait()
                pltpu.sync_copy(src_hbm.at[pl.ds(j1 + 2 * CH, CH)], sidx1)
                pltpu.async_copy(h_hbm.at[sidx1], rows1, gsem1)
            return carry

        lax.fori_loop(0, NFULL // 2, body, 0)
        pltpu.make_async_copy(rows0, acc.at[didx0], ssem0).wait()
        pltpu.make_async_copy(rows1, acc.at[didx1], ssem1).wait()

        if TAIL:
            toff = base + NFULL * CH
            pltpu.sync_copy(src_hbm.at[pl.ds(toff, TAIL)], tsidx)
            pltpu.sync_copy(dst_hbm.at[pl.ds(toff, TAIL)], tdidx)
            if with_deg:
                for k in range(TAIL // 16):
                    dv = tdidx[pl.ds(k * 16, 16)]
                    plsc.addupdate_scatter(hist, [dv], ones)
            pltpu.async_copy(h_hbm.at[tsidx], trows, gsem0).wait()
            pltpu.sync_copy(trows, acc.at[tdidx], add=True)

        if with_deg:
            pltpu.sync_copy(hist, stage_hbm.at[wid])
            plsc.subcore_barrier()

            def zdeg(j, carry):
                degv[pl.ds(j * 16, 16)] = jnp.zeros((16,), _f32)
                return carry
            lax.fori_loop(0, RPS // 16, zdeg, 0)
            for r in range(NS):
                pltpu.sync_copy(
                    stage_hbm.at[r * NC + c, pl.ds(s * RPS, RPS)], drow)

                def dbody(j, carry):
                    col = j * 16
                    degv[pl.ds(col, 16)] += drow[pl.ds(col, 16)]
                    return carry
                lax.fori_loop(0, RPS // 16, dbody, 0)
            pltpu.sync_copy(degv, deg_hbm.at[c, pl.ds(s * RPS, RPS)])

        plsc.subcore_barrier()
        pltpu.sync_copy(
            acc.at[pl.ds(s * RPS, RPS)],
            out_hbm.at[pl.ds(c * NPAD + s * RPS, RPS)])

    return agg


_sc_agg_deg = _make_sc_agg(True)
_sc_agg = _make_sc_agg(False)

BS = 400
NBLK = N // BS


def _convA_body(h_ref, part_ref, deg_ref, wr_ref, wn_ref, b_ref,
                hpre_ref, sum_ref, ssq_ref):
    i = pl.program_id(0)
    agg = part_ref[0] + part_ref[1]
    mean = agg / jnp.maximum(deg_ref[...], 1.0)
    hp = (jnp.dot(h_ref[...], wr_ref[...], preferred_element_type=_f32)
          + jnp.dot(mean, wn_ref[...], preferred_element_type=_f32)
          + b_ref[...])
    hpre_ref[...] = hp

    @pl.when(i == 0)
    def _():
        sum_ref[...] = jnp.zeros_like(sum_ref)
        ssq_ref[...] = jnp.zeros_like(ssq_ref)

    sum_ref[...] += jnp.sum(hp, axis=0, keepdims=True)
    ssq_ref[...] += jnp.sum(hp * hp, axis=0, keepdims=True)


def _convA(h, part, deg, wr, wn, b):
    return pl.pallas_call(
        _convA_body,
        grid=(NBLK,),
        in_specs=[
            pl.BlockSpec((BS, D), lambda i: (i, 0)),
            pl.BlockSpec((NC, BS, D), lambda i: (0, i, 0)),
            pl.BlockSpec((BS, 1), lambda i: (i, 0)),
            pl.BlockSpec((D, D), lambda i: (0, 0)),
            pl.BlockSpec((D, D), lambda i: (0, 0)),
            pl.BlockSpec((1, D), lambda i: (0, 0)),
        ],
        out_specs=[
            pl.BlockSpec((BS, D), lambda i: (i, 0)),
            pl.BlockSpec((1, D), lambda i: (0, 0)),
            pl.BlockSpec((1, D), lambda i: (0, 0)),
        ],
        out_shape=[
            jax.ShapeDtypeStruct((N, D), _f32),
            jax.ShapeDtypeStruct((1, D), _f32),
            jax.ShapeDtypeStruct((1, D), _f32),
        ],
    )(h, part, deg, wr, wn, b)


def _convB_body(hpre_ref, sum_ref, ssq_ref, gamma_ref, beta_ref, out_ref):
    mu = sum_ref[...] / N
    var = ssq_ref[...] / N - mu * mu
    rstd = lax.rsqrt(var + 1e-5)
    hb = (hpre_ref[...] - mu) * (rstd * gamma_ref[...]) + beta_ref[...]
    out_ref[...] = jnp.maximum(hb, 0.0)


def _convB(hpre, sm, sq, gamma, beta):
    return pl.pallas_call(
        _convB_body,
        grid=(NBLK,),
        in_specs=[
            pl.BlockSpec((BS, D), lambda i: (i, 0)),
            pl.BlockSpec((1, D), lambda i: (0, 0)),
            pl.BlockSpec((1, D), lambda i: (0, 0)),
            pl.BlockSpec((1, D), lambda i: (0, 0)),
            pl.BlockSpec((1, D), lambda i: (0, 0)),
        ],
        out_specs=pl.BlockSpec((BS, D), lambda i: (i, 0)),
        out_shape=jax.ShapeDtypeStruct((N, D), _f32),
    )(hpre, sm, sq, gamma, beta)


def _head_body(xn_ref, w1_ref, b1_ref, w2_ref, b2_ref, w3_ref, b3_ref,
               gsw1_ref, gsb1_ref, gsw2_ref, gsb2_ref,
               ghw1_ref, ghb1_ref, ghw2_ref, ghb2_ref, ghw3_ref, ghb3_ref,
               gout_ref, nout_ref):
    xn = xn_ref[...]
    h1 = lax.dot_general(xn, w1_ref[...], (((2,), (1,)), ((0,), (0,))),
                         preferred_element_type=_f32)
    h1 = jnp.maximum(h1 + b1_ref[...][:, None, :], 0.0)
    h2 = lax.dot_general(h1, w2_ref[...], (((2,), (1,)), ((0,), (0,))),
                         preferred_element_type=_f32)
    h2 = jnp.maximum(h2 + b2_ref[...][:, None, :], 0.0)
    w3 = w3_ref[...][:, :, 0]
    nout_ref[...] = jnp.sum(h2 * w3[:, None, :], axis=2) + b3_ref[...]

    g = jnp.maximum(jnp.sum(xn, axis=0) / NN, 0.0)
    g = jnp.dot(g, gsw1_ref[...], preferred_element_type=_f32) + gsb1_ref[...]
    g = jnp.dot(g, gsw2_ref[...], preferred_element_type=_f32) + gsb2_ref[...]
    g = jnp.maximum(g, 0.0)
    g = jnp.maximum(
        jnp.dot(g, ghw1_ref[...], preferred_element_type=_f32) + ghb1_ref[...], 0.0)
    g = jnp.maximum(
        jnp.dot(g, ghw2_ref[...], preferred_element_type=_f32) + ghb2_ref[...], 0.0)
    gout_ref[...] = (jnp.dot(g, ghw3_ref[...], preferred_element_type=_f32)
                     + ghb3_ref[...])


def _head(xnT, nh_W1, nh_b1, nh_W2, nh_b2, nh_W3, nh_b3,
          gs_W1, gs_b1, gs_W2, gs_b2,
          gh_W1, gh_b1, gh_W2, gh_b2, gh_W3, gh_b3):
    return pl.pallas_call(
        _head_body,
        out_shape=[
            jax.ShapeDtypeStruct((B, 1), _f32),
            jax.ShapeDtypeStruct((NN, B), _f32),
        ],
    )(xnT, nh_W1, nh_b1, nh_W2, nh_b2, nh_W3, nh_b3,
      gs_W1, gs_b1, gs_W2, gs_b2, gh_W1, gh_b1, gh_W2, gh_b2, gh_W3, gh_b3)


def kernel(x, conv_Wr, conv_Wn, conv_b, bn_gamma, bn_beta,
           gs_W1, gs_b1, gs_W2, gs_b2,
           gh_W1, gh_b1, gh_W2, gh_b2, gh_W3, gh_b3,
           nh_W1, nh_b1, nh_W2, nh_b2, nh_W3, nh_b3,
           edge_index, batch):
    src = edge_index[0]
    dst = edge_index[1]

    part0_flat, degp, _ = _sc_agg_deg(x, src, dst)
    part0 = part0_flat.reshape(NC, NPAD, D)
    deg = (degp[0] + degp[1])[:N].reshape(N, 1)
    hpre0, sm0, sq0 = _convA(x, part0, deg, conv_Wr[0], conv_Wn[0],
                             conv_b[0].reshape(1, D))
    h1 = _convB(hpre0, sm0, sq0, bn_gamma[0:1], bn_beta[0:1])

    part1 = _sc_agg(h1, src, dst)[0].reshape(NC, NPAD, D)
    hpre1, sm1, sq1 = _convA(h1, part1, deg, conv_Wr[1], conv_Wn[1],
                             conv_b[1].reshape(1, D))
    h2 = _convB(hpre1, sm1, sq1, bn_gamma[1:2], bn_beta[1:2])

    xnT = h2.reshape(B, NN, D).transpose(1, 0, 2)
    g_out, n_outT = _head(
        xnT, nh_W1, nh_b1, nh_W2, nh_b2, nh_W3, nh_b3,
        gs_W1, gs_b1.reshape(1, DS), gs_W2, gs_b2.reshape(1, DS),
        gh_W1, gh_b1.reshape(1, DH), gh_W2, gh_b2.reshape(1, DH),
        gh_W3, gh_b3.reshape(1, 1))
    return jnp.concatenate([g_out, n_outT.T], axis=1)
